# baseline (speedup 1.0000x reference)
import functools

import jax
import jax.numpy as jnp
from jax import lax
from jax.experimental import pallas as pl
from jax.experimental.pallas import tpu as pltpu
from jax.experimental.pallas import tpu_sc as plsc

_D = 96
_CHUNK = 512
_NBUF = 2


def _mlp_table_body(emb_ref, w1_ref, b1_ref, w2_ref, b2_ref, w4_ref, b4_ref,
                    out_ref):
    h = jnp.dot(emb_ref[...], w1_ref[...],
                preferred_element_type=jnp.float32) + b1_ref[...]
    h = jnp.maximum(h, 0.0)
    h = jnp.dot(h, w2_ref[...], preferred_element_type=jnp.float32) + b2_ref[...]
    h = jnp.maximum(h, 0.0)
    out_ref[...] = (jnp.dot(h, w4_ref[...], preferred_element_type=jnp.float32)
                    + b4_ref[...])


def _mlp_table(emb_p, W1, b1, W2, b2, W4, b4):
    return pl.pallas_call(
        _mlp_table_body,
        out_shape=jax.ShapeDtypeStruct((16, _D), jnp.float32),
    )(emb_p, W1, b1.reshape(1, _D), W2, b2.reshape(1, _D),
      W4, b4.reshape(1, _D))


@functools.partial(jax.jit, static_argnums=(2,))
def _sc_gather(table_flat, idx_flat, total_rows):
    info = plsc.get_sparse_core_info()
    nw = info.num_cores * info.num_subcores
    nc = info.num_cores
    L = info.num_lanes
    assert total_rows % (nw * _CHUNK * _NBUF) == 0
    b_per_w = total_rows // nw
    n_chunks = b_per_w // _CHUNK
    groups = _CHUNK // L

    mesh = plsc.VectorSubcoreMesh(core_axis_name="c", subcore_axis_name="s")

    @functools.partial(
        pl.kernel,
        mesh=mesh,
        compiler_params=pltpu.CompilerParams(needs_layout_passes=False),
        out_type=jax.ShapeDtypeStruct((total_rows * _D,), jnp.float32),
        scratch_types=[
            pltpu.VMEM((16 * 128,), jnp.float32),
            pltpu.VMEM((b_per_w,), jnp.int32),
            pltpu.VMEM((_CHUNK * _D,), jnp.float32),
            pltpu.VMEM((_CHUNK * _D,), jnp.float32),
            pltpu.SemaphoreType.DMA,
            pltpu.SemaphoreType.DMA,
        ],
    )
    def k(table_hbm, idx_hbm, out_hbm, tbl_v, idx_v, rows0, rows1,
          sem0, sem1):
        wid = lax.axis_index("s") * nc + lax.axis_index("c")
        base = pl.multiple_of(wid * b_per_w, _CHUNK)
        pltpu.sync_copy(table_hbm, tbl_v)
        pltpu.sync_copy(idx_hbm.at[pl.ds(base, b_per_w)], idx_v)
        rows = [rows0, rows1]
        sems = [sem0, sem1]
        posc = lax.iota(jnp.int32, L) * _D

        def outer(t, carry):
            for b in range(_NBUF):
                kk = t * _NBUF + b
                off = pl.multiple_of(base + kk * _CHUNK, _CHUNK)
                dst = out_hbm.at[pl.ds(off * _D, _CHUNK * _D)]

                @pl.when(t > 0)
                def _wait(b=b, dst=dst):
                    pltpu.make_async_copy(rows[b], dst, sems[b]).wait()

                def group(g, c2, kk=kk, b=b):
                    p0 = pl.multiple_of(kk * _CHUNK + g * L, L)
                    idx16 = idx_v[pl.ds(p0, L)]
                    addr = idx16 * 128
                    posg = posc + g * (L * _D)
                    for d in range(_D):
                        v = plsc.load_gather(tbl_v, [addr + d])
                        plsc.store_scatter(rows[b], [posg + d], v)
                    return c2

                lax.fori_loop(0, groups, group, 0)
                pltpu.async_copy(rows[b], dst, sems[b])
            return carry

        lax.fori_loop(0, n_chunks // _NBUF, outer, 0)
        for b in range(_NBUF):
            last = pl.multiple_of(
                base + (n_chunks - _NBUF + b) * _CHUNK, _CHUNK)
            pltpu.make_async_copy(
                rows[b], out_hbm.at[pl.ds(last * _D, _CHUNK * _D)],
                sems[b]).wait()

    return k(table_flat, idx_flat)


def kernel(x, r, c, emb_input, emb_row, emb_col, W1, b1, W2, b2, W4, b4):
    del r, c, emb_row, emb_col
    n, s = x.shape
    total = n * s
    emb_p = jnp.zeros((16, 16), jnp.float32).at[:emb_input.shape[0]].set(
        emb_input)
    table = _mlp_table(emb_p, W1, b1, W2, b2, W4, b4)
    table_p = jnp.zeros((16, 128), jnp.float32).at[:, :_D].set(table)
    idx_flat = x.astype(jnp.int32).reshape(total)
    out = _sc_gather(table_p.reshape(16 * 128), idx_flat, total)
    return out.reshape(n, s, _D)


# speedup vs baseline: 2.9978x; 2.9978x over previous
import functools

import jax
import jax.numpy as jnp
from jax import lax
from jax.experimental import pallas as pl
from jax.experimental.pallas import tpu as pltpu
from jax.experimental.pallas import tpu_sc as plsc

_D = 96
_CHUNK = 512
_NBUF = 2


def _mlp_table_body(emb_ref, w1_ref, b1_ref, w2_ref, b2_ref, w4_ref, b4_ref,
                    out_ref):
    h = jnp.dot(emb_ref[...], w1_ref[...],
                preferred_element_type=jnp.float32) + b1_ref[...]
    h = jnp.maximum(h, 0.0)
    h = jnp.dot(h, w2_ref[...], preferred_element_type=jnp.float32) + b2_ref[...]
    h = jnp.maximum(h, 0.0)
    out_ref[...] = (jnp.dot(h, w4_ref[...], preferred_element_type=jnp.float32)
                    + b4_ref[...])


def _mlp_table(emb_p, W1, b1, W2, b2, W4, b4):
    return pl.pallas_call(
        _mlp_table_body,
        out_shape=jax.ShapeDtypeStruct((16, _D), jnp.float32),
    )(emb_p, W1, b1.reshape(1, _D), W2, b2.reshape(1, _D),
      W4, b4.reshape(1, _D))


@functools.partial(jax.jit, static_argnums=(2,))
def _sc_gather(table_flat, idx_flat, total_rows):
    info = plsc.get_sparse_core_info()
    nw = info.num_cores * info.num_subcores
    nc = info.num_cores
    L = info.num_lanes
    assert total_rows % (nw * _CHUNK * _NBUF) == 0
    b_per_w = total_rows // nw
    n_chunks = b_per_w // _CHUNK
    groups = _CHUNK // L

    mesh = plsc.VectorSubcoreMesh(core_axis_name="c", subcore_axis_name="s")

    @functools.partial(
        pl.kernel,
        mesh=mesh,
        compiler_params=pltpu.CompilerParams(needs_layout_passes=False),
        out_type=jax.ShapeDtypeStruct((total_rows * _D,), jnp.float32),
        scratch_types=[
            pltpu.VMEM((16 * 128,), jnp.float32),
            pltpu.VMEM((b_per_w,), jnp.int32),
            pltpu.VMEM((_CHUNK * _D,), jnp.float32),
            pltpu.VMEM((_CHUNK * _D,), jnp.float32),
            pltpu.SemaphoreType.DMA,
            pltpu.SemaphoreType.DMA,
        ],
    )
    def k(table_hbm, idx_hbm, out_hbm, tbl_v, idx_v, rows0, rows1,
          sem0, sem1):
        wid = lax.axis_index("s") * nc + lax.axis_index("c")
        base = pl.multiple_of(wid * b_per_w, _CHUNK)
        pltpu.sync_copy(table_hbm, tbl_v)
        pltpu.sync_copy(idx_hbm.at[pl.ds(base, b_per_w)], idx_v)
        rows = [rows0, rows1]
        sems = [sem0, sem1]
        lane = lax.iota(jnp.int32, L)
        cols = [lane + j * L for j in range(_D // L)]
        splats = [jnp.full((L,), l, jnp.int32) for l in range(L)]

        def outer(t, carry):
            for b in range(_NBUF):
                kk = t * _NBUF + b
                off = pl.multiple_of(base + kk * _CHUNK, _CHUNK)
                dst = out_hbm.at[pl.ds(off * _D, _CHUNK * _D)]

                @pl.when(t > 0)
                def _wait(b=b, dst=dst):
                    pltpu.make_async_copy(rows[b], dst, sems[b]).wait()

                def group(g, c2, kk=kk, b=b):
                    p0 = pl.multiple_of(kk * _CHUNK + g * L, L)
                    idx16 = idx_v[pl.ds(p0, L)]
                    addr16 = idx16 * 128
                    for l in range(L):
                        rowa = jnp.take_along_axis(addr16, splats[l], axis=0)
                        dstb = pl.multiple_of(g * (L * _D) + l * _D, 8)
                        for j in range(_D // L):
                            v = plsc.load_gather(tbl_v, [rowa + cols[j]])
                            rows[b][pl.ds(dstb + j * L, L)] = v
                    return c2

                lax.fori_loop(0, groups, group, 0)
                pltpu.async_copy(rows[b], dst, sems[b])
            return carry

        lax.fori_loop(0, n_chunks // _NBUF, outer, 0)
        for b in range(_NBUF):
            last = pl.multiple_of(
                base + (n_chunks - _NBUF + b) * _CHUNK, _CHUNK)
            pltpu.make_async_copy(
                rows[b], out_hbm.at[pl.ds(last * _D, _CHUNK * _D)],
                sems[b]).wait()

    return k(table_flat, idx_flat)


def kernel(x, r, c, emb_input, emb_row, emb_col, W1, b1, W2, b2, W4, b4):
    del r, c, emb_row, emb_col
    n, s = x.shape
    total = n * s
    emb_p = jnp.zeros((16, 16), jnp.float32).at[:emb_input.shape[0]].set(
        emb_input)
    table = _mlp_table(emb_p, W1, b1, W2, b2, W4, b4)
    table_p = jnp.zeros((16, 128), jnp.float32).at[:, :_D].set(table)
    idx_flat = x.astype(jnp.int32).reshape(total)
    out = _sc_gather(table_p.reshape(16 * 128), idx_flat, total)
    return out.reshape(n, s, _D)


# speedup vs baseline: 4.0575x; 1.3535x over previous
import functools

import jax
import jax.numpy as jnp
from jax import lax
from jax.experimental import pallas as pl
from jax.experimental.pallas import tpu as pltpu
from jax.experimental.pallas import tpu_sc as plsc

_D = 96
_CHUNK = 512
_NBUF = 2


def _mlp_table_body(emb_ref, w1_ref, b1_ref, w2_ref, b2_ref, w4_ref, b4_ref,
                    out_ref):
    h = jnp.dot(emb_ref[...], w1_ref[...],
                preferred_element_type=jnp.float32) + b1_ref[...]
    h = jnp.maximum(h, 0.0)
    h = jnp.dot(h, w2_ref[...], preferred_element_type=jnp.float32) + b2_ref[...]
    h = jnp.maximum(h, 0.0)
    out_ref[...] = (jnp.dot(h, w4_ref[...], preferred_element_type=jnp.float32)
                    + b4_ref[...])


def _mlp_table(emb_p, W1, b1, W2, b2, W4, b4):
    return pl.pallas_call(
        _mlp_table_body,
        out_shape=jax.ShapeDtypeStruct((16, _D), jnp.float32),
    )(emb_p, W1, b1.reshape(1, _D), W2, b2.reshape(1, _D),
      W4, b4.reshape(1, _D))


_SLABS = 8
_CROWS = _SLABS * 50


@functools.partial(jax.jit, static_argnums=(2, 3))
def _sc_gather(table_flat, idx_flat, n_slab, seq):
    info = plsc.get_sparse_core_info()
    nw = info.num_cores * info.num_subcores
    nc = info.num_cores
    L = info.num_lanes
    assert n_slab % (nw * _SLABS * _NBUF) == 0 and seq == 50
    slab_per_w = n_slab // nw
    b_per_w = slab_per_w * seq
    n_chunks = slab_per_w // _SLABS
    groups = _CROWS // L

    mesh = plsc.VectorSubcoreMesh(core_axis_name="c", subcore_axis_name="s")

    @functools.partial(
        pl.kernel,
        mesh=mesh,
        compiler_params=pltpu.CompilerParams(needs_layout_passes=False),
        out_type=jax.ShapeDtypeStruct((n_slab, seq, _D), jnp.float32),
        scratch_types=[
            pltpu.VMEM((16 * 128,), jnp.float32),
            pltpu.VMEM((b_per_w,), jnp.int32),
            pltpu.VMEM((_CROWS, _D), jnp.float32),
            pltpu.VMEM((_CROWS, _D), jnp.float32),
            pltpu.SemaphoreType.DMA,
            pltpu.SemaphoreType.DMA,
        ],
    )
    def k(table_hbm, idx_hbm, out_hbm, tbl_v, idx_v, rows0, rows1,
          sem0, sem1):
        wid = lax.axis_index("s") * nc + lax.axis_index("c")
        base = pl.multiple_of(wid * b_per_w, _CROWS)
        sbase = pl.multiple_of(wid * slab_per_w, _SLABS)
        pltpu.sync_copy(table_hbm, tbl_v)
        pltpu.sync_copy(idx_hbm.at[pl.ds(base, b_per_w)], idx_v)
        rows = [rows0, rows1]
        sems = [sem0, sem1]
        lane = lax.iota(jnp.int32, L)
        cols = [lane + j * L for j in range(_D // L)]
        splats = [jnp.full((L,), l, jnp.int32) for l in range(L)]

        def slab_copies(b, slab0, issue):
            for s in range(_SLABS):
                cp = pltpu.make_async_copy(
                    rows[b].at[pl.ds(s * seq, seq)],
                    out_hbm.at[slab0 + s], sems[b])
                if issue:
                    cp.start()
                else:
                    cp.wait()

        def outer(t, carry):
            for b in range(_NBUF):
                kk = t * _NBUF + b
                slab0 = pl.multiple_of(sbase + kk * _SLABS, _SLABS)

                @pl.when(t > 0)
                def _wait(b=b, slab0=slab0):
                    slab_copies(b, slab0, issue=False)

                def group(g, c2, kk=kk, b=b):
                    p0 = pl.multiple_of(kk * _CROWS + g * L, L)
                    idx16 = idx_v[pl.ds(p0, L)]
                    addr16 = idx16 * 128
                    for l in range(L):
                        rowa = jnp.take_along_axis(addr16, splats[l], axis=0)
                        r = pl.multiple_of(g * L + l, 1)
                        for j in range(_D // L):
                            v = plsc.load_gather(tbl_v, [rowa + cols[j]])
                            rows[b][r, pl.ds(j * L, L)] = v
                    return c2

                lax.fori_loop(0, groups, group, 0)
                slab_copies(b, slab0, issue=True)
            return carry

        lax.fori_loop(0, n_chunks // _NBUF, outer, 0)
        for b in range(_NBUF):
            last = pl.multiple_of(
                sbase + (n_chunks - _NBUF + b) * _SLABS, _SLABS)
            slab_copies(b, last, issue=False)

    return k(table_flat, idx_flat)


def kernel(x, r, c, emb_input, emb_row, emb_col, W1, b1, W2, b2, W4, b4):
    del r, c, emb_row, emb_col
    n, s = x.shape
    total = n * s
    emb_p = jnp.zeros((16, 16), jnp.float32).at[:emb_input.shape[0]].set(
        emb_input)
    table = _mlp_table(emb_p, W1, b1, W2, b2, W4, b4)
    table_p = jnp.zeros((16, 128), jnp.float32).at[:, :_D].set(table)
    idx_flat = x.astype(jnp.int32).reshape(total)
    return _sc_gather(table_p.reshape(16 * 128), idx_flat, n, s)


# speedup vs baseline: 4.0579x; 1.0001x over previous
import functools

import jax
import jax.numpy as jnp
from jax import lax
from jax.experimental import pallas as pl
from jax.experimental.pallas import tpu as pltpu
from jax.experimental.pallas import tpu_sc as plsc

_D = 96
_CHUNK = 512
_NBUF = 2


def _mlp_table_body(emb_ref, w1_ref, b1_ref, w2_ref, b2_ref, w4_ref, b4_ref,
                    out_ref):
    h = jnp.dot(emb_ref[...], w1_ref[...],
                preferred_element_type=jnp.float32) + b1_ref[...]
    h = jnp.maximum(h, 0.0)
    h = jnp.dot(h, w2_ref[...], preferred_element_type=jnp.float32) + b2_ref[...]
    h = jnp.maximum(h, 0.0)
    out_ref[...] = (jnp.dot(h, w4_ref[...], preferred_element_type=jnp.float32)
                    + b4_ref[...])


def _mlp_table(emb_p, W1, b1, W2, b2, W4, b4):
    return pl.pallas_call(
        _mlp_table_body,
        out_shape=jax.ShapeDtypeStruct((16, _D), jnp.float32),
    )(emb_p, W1, b1.reshape(1, _D), W2, b2.reshape(1, _D),
      W4, b4.reshape(1, _D))


_SLABS = 8
_CROWS = _SLABS * 50


@functools.partial(jax.jit, static_argnums=(2, 3))
def _sc_gather(table_flat, idx_flat, n_slab, seq):
    info = plsc.get_sparse_core_info()
    nw = info.num_cores * info.num_subcores
    nc = info.num_cores
    L = info.num_lanes
    assert n_slab % (nw * _SLABS * _NBUF) == 0 and seq == 50
    slab_per_w = n_slab // nw
    b_per_w = slab_per_w * seq
    n_chunks = slab_per_w // _SLABS
    groups = _CROWS // L

    mesh = plsc.VectorSubcoreMesh(core_axis_name="c", subcore_axis_name="s")

    @functools.partial(
        pl.kernel,
        mesh=mesh,
        compiler_params=pltpu.CompilerParams(needs_layout_passes=False),
        out_type=jax.ShapeDtypeStruct((n_slab, seq, _D), jnp.float32),
        scratch_types=[
            pltpu.VMEM((16, 128), jnp.float32),
            pltpu.VMEM((b_per_w,), jnp.int32),
            pltpu.VMEM((_CROWS, _D), jnp.float32),
            pltpu.VMEM((_CROWS, _D), jnp.float32),
            pltpu.SemaphoreType.DMA,
            pltpu.SemaphoreType.DMA,
        ],
    )
    def k(table_hbm, idx_hbm, out_hbm, tbl_v, idx_v, rows0, rows1,
          sem0, sem1):
        wid = lax.axis_index("s") * nc + lax.axis_index("c")
        base = pl.multiple_of(wid * b_per_w, _CROWS)
        sbase = pl.multiple_of(wid * slab_per_w, _SLABS)
        pltpu.sync_copy(table_hbm, tbl_v)
        pltpu.sync_copy(idx_hbm.at[pl.ds(base, b_per_w)], idx_v)
        rows = [rows0, rows1]
        sems = [sem0, sem1]
        lane = lax.iota(jnp.int32, L)
        cols = [lane + j * L for j in range(_D // L)]
        splats = [jnp.full((L,), l, jnp.int32) for l in range(L)]

        def slab_copies(b, slab0, issue):
            for s in range(_SLABS):
                cp = pltpu.make_async_copy(
                    rows[b].at[pl.ds(s * seq, seq)],
                    out_hbm.at[slab0 + s], sems[b])
                if issue:
                    cp.start()
                else:
                    cp.wait()

        def outer(t, carry):
            for b in range(_NBUF):
                kk = t * _NBUF + b
                slab0 = pl.multiple_of(sbase + kk * _SLABS, _SLABS)

                @pl.when(t > 0)
                def _wait(b=b, slab0=slab0):
                    slab_copies(b, slab0, issue=False)

                def group(g, c2, kk=kk, b=b):
                    p0 = pl.multiple_of(kk * _CROWS + g * L, L)
                    idx16 = idx_v[pl.ds(p0, L)]
                    for l in range(L):
                        rowv = jnp.take_along_axis(idx16, splats[l], axis=0)
                        r = g * L + l
                        for j in range(_D // L):
                            v = plsc.load_gather(tbl_v, [rowv, cols[j]])
                            rows[b][r, pl.ds(j * L, L)] = v
                    return c2

                lax.fori_loop(0, groups, group, 0)
                slab_copies(b, slab0, issue=True)
            return carry

        lax.fori_loop(0, n_chunks // _NBUF, outer, 0)
        for b in range(_NBUF):
            last = pl.multiple_of(
                sbase + (n_chunks - _NBUF + b) * _SLABS, _SLABS)
            slab_copies(b, last, issue=False)

    return k(table_flat, idx_flat)


def kernel(x, r, c, emb_input, emb_row, emb_col, W1, b1, W2, b2, W4, b4):
    del r, c, emb_row, emb_col
    n, s = x.shape
    total = n * s
    emb_p = jnp.zeros((16, 16), jnp.float32).at[:emb_input.shape[0]].set(
        emb_input)
    table = _mlp_table(emb_p, W1, b1, W2, b2, W4, b4)
    table_p = jnp.zeros((16, 128), jnp.float32).at[:, :_D].set(table)
    idx_flat = x.astype(jnp.int32).reshape(total)
    return _sc_gather(table_p, idx_flat, n, s)


# speedup vs baseline: 6.6375x; 1.6357x over previous
import functools

import jax
import jax.numpy as jnp
from jax import lax
from jax.experimental import pallas as pl
from jax.experimental.pallas import tpu as pltpu
from jax.experimental.pallas import tpu_sc as plsc

_D = 96
_CHUNK = 512
_NBUF = 2


def _mlp_table_body(emb_ref, w1_ref, b1_ref, w2_ref, b2_ref, w4_ref, b4_ref,
                    out_ref):
    h = jnp.dot(emb_ref[...], w1_ref[...],
                preferred_element_type=jnp.float32) + b1_ref[...]
    h = jnp.maximum(h, 0.0)
    h = jnp.dot(h, w2_ref[...], preferred_element_type=jnp.float32) + b2_ref[...]
    h = jnp.maximum(h, 0.0)
    out_ref[...] = (jnp.dot(h, w4_ref[...], preferred_element_type=jnp.float32)
                    + b4_ref[...])


def _mlp_table(emb_p, W1, b1, W2, b2, W4, b4):
    return pl.pallas_call(
        _mlp_table_body,
        out_shape=jax.ShapeDtypeStruct((16, _D), jnp.float32),
    )(emb_p, W1, b1.reshape(1, _D), W2, b2.reshape(1, _D),
      W4, b4.reshape(1, _D))


_SLABS = 8
_CROWS = _SLABS * 50


@functools.partial(jax.jit, static_argnums=(2, 3))
def _sc_gather(table_flat, idx_flat, n_slab, seq):
    info = plsc.get_sparse_core_info()
    nw = info.num_cores * info.num_subcores
    nc = info.num_cores
    L = info.num_lanes
    assert n_slab % (nw * _SLABS * _NBUF) == 0 and seq == 50
    slab_per_w = n_slab // nw
    b_per_w = slab_per_w * seq
    n_chunks = slab_per_w // _SLABS
    groups = _CROWS // L

    mesh = plsc.VectorSubcoreMesh(core_axis_name="c", subcore_axis_name="s")

    @functools.partial(
        pl.kernel,
        mesh=mesh,
        compiler_params=pltpu.CompilerParams(needs_layout_passes=False),
        out_type=jax.ShapeDtypeStruct((n_slab, seq, _D), jnp.float32),
        scratch_types=[
            pltpu.VMEM((16, 128), jnp.float32),
            pltpu.VMEM((b_per_w,), jnp.int32),
            pltpu.VMEM((_CROWS, _D), jnp.float32),
            pltpu.VMEM((_CROWS, _D), jnp.float32),
            pltpu.SemaphoreType.DMA,
            pltpu.SemaphoreType.DMA,
        ],
    )
    def k(table_hbm, idx_hbm, out_hbm, tbl_v, idx_v, rows0, rows1,
          sem0, sem1):
        wid = lax.axis_index("s") * nc + lax.axis_index("c")
        base = pl.multiple_of(wid * b_per_w, _CROWS)
        sbase = pl.multiple_of(wid * slab_per_w, _SLABS)
        pltpu.sync_copy(table_hbm, tbl_v)
        pltpu.sync_copy(idx_hbm.at[pl.ds(base, b_per_w)], idx_v)
        rows = [rows0, rows1]
        sems = [sem0, sem1]
        lane = lax.iota(jnp.int32, L)
        cols = [lane + j * L for j in range(_D // L)]
        splats = [jnp.full((L,), l, jnp.int32) for l in range(L)]

        def slab_copies(b, slab0, issue):
            for s in range(_SLABS):
                cp = pltpu.make_async_copy(
                    rows[b].at[pl.ds(s * seq, seq)],
                    out_hbm.at[slab0 + s], sems[b])
                if issue:
                    cp.start()
                else:
                    cp.wait()

        def outer(t, carry):
            for b in range(_NBUF):
                kk = t * _NBUF + b
                slab0 = pl.multiple_of(sbase + kk * _SLABS, _SLABS)

                @pl.when(t > 0)
                def _wait(b=b, slab0=slab0):
                    slab_copies(b, slab0, issue=False)

                @plsc.parallel_loop(0, groups, unroll=1)
                def group(g, kk=kk, b=b):
                    p0 = pl.multiple_of(kk * _CROWS + g * L, L)
                    idx16 = idx_v[pl.ds(p0, L)]
                    for l in range(L):
                        rowv = jnp.take_along_axis(idx16, splats[l], axis=0)
                        r = g * L + l
                        for j in range(_D // L):
                            v = plsc.load_gather(tbl_v, [rowv, cols[j]])
                            rows[b][r, pl.ds(j * L, L)] = v
                slab_copies(b, slab0, issue=True)
            return carry

        lax.fori_loop(0, n_chunks // _NBUF, outer, 0)
        for b in range(_NBUF):
            last = pl.multiple_of(
                sbase + (n_chunks - _NBUF + b) * _SLABS, _SLABS)
            slab_copies(b, last, issue=False)

    return k(table_flat, idx_flat)


def kernel(x, r, c, emb_input, emb_row, emb_col, W1, b1, W2, b2, W4, b4):
    del r, c, emb_row, emb_col
    n, s = x.shape
    total = n * s
    emb_p = jnp.zeros((16, 16), jnp.float32).at[:emb_input.shape[0]].set(
        emb_input)
    table = _mlp_table(emb_p, W1, b1, W2, b2, W4, b4)
    table_p = jnp.zeros((16, 128), jnp.float32).at[:, :_D].set(table)
    idx_flat = x.astype(jnp.int32).reshape(total)
    return _sc_gather(table_p, idx_flat, n, s)


# speedup vs baseline: 9.0687x; 1.3663x over previous
import functools

import jax
import jax.numpy as jnp
from jax import lax
from jax.experimental import pallas as pl
from jax.experimental.pallas import tpu as pltpu
from jax.experimental.pallas import tpu_sc as plsc

_D = 96
_IC = 256
_SKEW = 17


def _mlp_table_body(emb_ref, w1_ref, b1_ref, w2_ref, b2_ref, w4_ref, b4_ref,
                    out_ref):
    h = jnp.dot(emb_ref[...], w1_ref[...],
                preferred_element_type=jnp.float32) + b1_ref[...]
    h = jnp.maximum(h, 0.0)
    h = jnp.dot(h, w2_ref[...], preferred_element_type=jnp.float32) + b2_ref[...]
    h = jnp.maximum(h, 0.0)
    out_ref[...] = (jnp.dot(h, w4_ref[...], preferred_element_type=jnp.float32)
                    + b4_ref[...])


def _mlp_table(emb_p, W1, b1, W2, b2, W4, b4):
    return pl.pallas_call(
        _mlp_table_body,
        out_shape=jax.ShapeDtypeStruct((16, _D), jnp.float32),
    )(emb_p, W1, b1.reshape(1, _D), W2, b2.reshape(1, _D),
      W4, b4.reshape(1, _D))


@functools.partial(jax.jit, static_argnums=(2, 3))
def _sc_gather(tbl_sk, idx_t, n, seq):
    info = plsc.get_sparse_core_info()
    nw = info.num_cores * info.num_subcores
    nc = info.num_cores
    L = info.num_lanes
    assert n % (nw * 2 * _IC) == 0
    i_per_w = n // nw
    groups = _IC // L
    drow = 16 * _SKEW

    mesh = plsc.VectorSubcoreMesh(core_axis_name="c", subcore_axis_name="s")

    @functools.partial(
        pl.kernel,
        mesh=mesh,
        compiler_params=pltpu.CompilerParams(needs_layout_passes=False),
        out_type=jax.ShapeDtypeStruct((seq, _D, n), jnp.float32),
        scratch_types=[
            pltpu.VMEM((_D * drow,), jnp.float32),
            pltpu.VMEM((seq, i_per_w), jnp.int32),
            pltpu.VMEM((_D, _IC), jnp.float32),
            pltpu.VMEM((_D, _IC), jnp.float32),
            pltpu.SemaphoreType.DMA,
            pltpu.SemaphoreType.DMA,
        ],
    )
    def k(tbl_hbm, idx_hbm, out_hbm, tbl_v, idx_v, rows0, rows1, sem0, sem1):
        wid = lax.axis_index("s") * nc + lax.axis_index("c")
        i0w = pl.multiple_of(wid * i_per_w, i_per_w)
        pltpu.sync_copy(tbl_hbm, tbl_v)
        pltpu.sync_copy(idx_hbm.at[:, pl.ds(i0w, i_per_w)], idx_v)
        rows = [rows0, rows1]
        sems = [sem0, sem1]
        laneoff = lax.iota(jnp.int32, L) * _SKEW

        def make_copy(b, j):
            return pltpu.make_async_copy(
                rows[b],
                out_hbm.at[j, :, pl.ds(pl.multiple_of(i0w + b * _IC, _IC),
                                       _IC)],
                sems[b])

        def jloop(j, carry):
            for b in range(2):
                @pl.when(j > 0)
                def _wait(b=b, j=j):
                    make_copy(b, j - 1).wait()

                @plsc.parallel_loop(0, groups, unroll=1)
                def group(g, b=b, j=j):
                    idx16 = idx_v[j, pl.ds(b * _IC + g * L, L)]
                    tmp = idx16 + laneoff
                    for d in range(_D):
                        v = plsc.load_gather(tbl_v, [tmp + d * drow])
                        rows[b][d, pl.ds(g * L, L)] = v

                make_copy(b, j).start()
            return carry

        lax.fori_loop(0, seq, jloop, 0)
        for b in range(2):
            make_copy(b, seq - 1).wait()

    return k(tbl_sk, idx_t)


def kernel(x, r, c, emb_input, emb_row, emb_col, W1, b1, W2, b2, W4, b4):
    del r, c, emb_row, emb_col
    n, s = x.shape
    emb_p = jnp.zeros((16, 16), jnp.float32).at[:emb_input.shape[0]].set(
        emb_input)
    table = _mlp_table(emb_p, W1, b1, W2, b2, W4, b4)
    tt = table.T
    pos = (jnp.arange(16)[:, None] * _SKEW
           + jnp.arange(16)[None, :]).reshape(-1)
    vals = jnp.broadcast_to(tt[:, None, :], (_D, 16, 16)).reshape(_D, 256)
    tbl_sk = jnp.zeros((_D, 16 * _SKEW), jnp.float32).at[:, pos].set(vals)
    idx_t = x.astype(jnp.int32).T
    out_t = _sc_gather(tbl_sk.reshape(-1), idx_t, n, s)
    return jnp.transpose(out_t, (2, 0, 1))


# speedup vs baseline: 9.7728x; 1.0776x over previous
import functools

import jax
import jax.numpy as jnp
from jax import lax
from jax.experimental import pallas as pl
from jax.experimental.pallas import tpu as pltpu
from jax.experimental.pallas import tpu_sc as plsc

_D = 96
_IC = 256


def _mlp_table_body(emb_ref, w1_ref, b1_ref, w2_ref, b2_ref, w4_ref, b4_ref,
                    out_ref):
    h = jnp.dot(emb_ref[...], w1_ref[...],
                preferred_element_type=jnp.float32) + b1_ref[...]
    h = jnp.maximum(h, 0.0)
    h = jnp.dot(h, w2_ref[...], preferred_element_type=jnp.float32) + b2_ref[...]
    h = jnp.maximum(h, 0.0)
    out_ref[...] = (jnp.dot(h, w4_ref[...], preferred_element_type=jnp.float32)
                    + b4_ref[...])


def _mlp_table(emb_p, W1, b1, W2, b2, W4, b4):
    return pl.pallas_call(
        _mlp_table_body,
        out_shape=jax.ShapeDtypeStruct((16, _D), jnp.float32),
    )(emb_p, W1, b1.reshape(1, _D), W2, b2.reshape(1, _D),
      W4, b4.reshape(1, _D))


@functools.partial(jax.jit, static_argnums=(2, 3))
def _sc_gather(tbl_sk, idx_t, n, seq):
    info = plsc.get_sparse_core_info()
    nw = info.num_cores * info.num_subcores
    nc = info.num_cores
    L = info.num_lanes
    assert n % (nw * 2 * _IC) == 0
    i_per_w = n // nw
    groups = _IC // L
    drow = 16 * L

    mesh = plsc.VectorSubcoreMesh(core_axis_name="c", subcore_axis_name="s")

    @functools.partial(
        pl.kernel,
        mesh=mesh,
        compiler_params=pltpu.CompilerParams(needs_layout_passes=False),
        out_type=jax.ShapeDtypeStruct((seq, _D, n), jnp.float32),
        scratch_types=[
            pltpu.VMEM((_D * drow,), jnp.float32),
            pltpu.VMEM((seq, i_per_w), jnp.int32),
            pltpu.VMEM((_D, _IC), jnp.float32),
            pltpu.VMEM((_D, _IC), jnp.float32),
            pltpu.SemaphoreType.DMA,
            pltpu.SemaphoreType.DMA,
        ],
    )
    def k(tbl_hbm, idx_hbm, out_hbm, tbl_v, idx_v, rows0, rows1, sem0, sem1):
        wid = lax.axis_index("s") * nc + lax.axis_index("c")
        i0w = pl.multiple_of(wid * i_per_w, i_per_w)
        pltpu.sync_copy(tbl_hbm, tbl_v)
        pltpu.sync_copy(idx_hbm.at[:, pl.ds(i0w, i_per_w)], idx_v)
        rows = [rows0, rows1]
        sems = [sem0, sem1]
        laneoff = lax.iota(jnp.int32, L)

        def make_copy(b, j):
            return pltpu.make_async_copy(
                rows[b],
                out_hbm.at[j, :, pl.ds(pl.multiple_of(i0w + b * _IC, _IC),
                                       _IC)],
                sems[b])

        def jloop(j, carry):
            for b in range(2):
                @pl.when(j > 0)
                def _wait(b=b, j=j):
                    make_copy(b, j - 1).wait()

                @plsc.parallel_loop(0, groups, unroll=1)
                def group(g, b=b, j=j):
                    idx16 = idx_v[j, pl.ds(b * _IC + g * L, L)]
                    tmp = idx16 * L + laneoff
                    for d in range(_D):
                        v = plsc.load_gather(tbl_v, [tmp + d * drow])
                        rows[b][d, pl.ds(g * L, L)] = v

                make_copy(b, j).start()
            return carry

        lax.fori_loop(0, seq, jloop, 0)
        for b in range(2):
            make_copy(b, seq - 1).wait()

    return k(tbl_sk, idx_t)


def kernel(x, r, c, emb_input, emb_row, emb_col, W1, b1, W2, b2, W4, b4):
    del r, c, emb_row, emb_col
    n, s = x.shape
    emb_p = jnp.zeros((16, 16), jnp.float32).at[:emb_input.shape[0]].set(
        emb_input)
    table = _mlp_table(emb_p, W1, b1, W2, b2, W4, b4)
    tt = table.T
    tbl_sk = jnp.broadcast_to(tt[:, :, None], (_D, 16, 16))
    idx_t = x.astype(jnp.int32).T
    out_t = _sc_gather(tbl_sk.reshape(-1), idx_t, n, s)
    return jnp.transpose(out_t, (2, 0, 1))


# speedup vs baseline: 23.5542x; 2.4102x over previous
import functools

import jax
import jax.numpy as jnp
from jax import lax
from jax.experimental import pallas as pl
from jax.experimental.pallas import tpu as pltpu
from jax.experimental.pallas import tpu_sc as plsc

_D = 96
_IC = 256


def _mlp_table_body(emb_ref, w1_ref, b1_ref, w2_ref, b2_ref, w4_ref, b4_ref,
                    out_ref):
    h = jnp.dot(emb_ref[...], w1_ref[...],
                preferred_element_type=jnp.float32) + b1_ref[...]
    h = jnp.maximum(h, 0.0)
    h = jnp.dot(h, w2_ref[...], preferred_element_type=jnp.float32) + b2_ref[...]
    h = jnp.maximum(h, 0.0)
    out_ref[...] = (jnp.dot(h, w4_ref[...], preferred_element_type=jnp.float32)
                    + b4_ref[...])


def _mlp_table(emb_p, W1, b1, W2, b2, W4, b4):
    return pl.pallas_call(
        _mlp_table_body,
        out_shape=jax.ShapeDtypeStruct((16, _D), jnp.float32),
    )(emb_p, W1, b1.reshape(1, _D), W2, b2.reshape(1, _D),
      W4, b4.reshape(1, _D))


@functools.partial(jax.jit, static_argnums=(2, 3))
def _sc_gather(tbl_sk, idx_t, n, seq):
    info = plsc.get_sparse_core_info()
    nw = info.num_cores * info.num_subcores
    nc = info.num_cores
    L = info.num_lanes
    assert n % (nw * 2 * _IC) == 0
    i_per_w = n // nw
    groups = _IC // L
    drow = 16 * L

    mesh = plsc.VectorSubcoreMesh(core_axis_name="c", subcore_axis_name="s")

    @functools.partial(
        pl.kernel,
        mesh=mesh,
        compiler_params=pltpu.CompilerParams(needs_layout_passes=False),
        out_type=jax.ShapeDtypeStruct((seq, _D, n), jnp.float32),
        scratch_types=[
            pltpu.VMEM((_D * drow,), jnp.float32),
            pltpu.VMEM((seq, i_per_w), jnp.int32),
            pltpu.VMEM((_D, _IC), jnp.float32),
            pltpu.VMEM((_D, _IC), jnp.float32),
            pltpu.SemaphoreType.DMA,
            pltpu.SemaphoreType.DMA,
        ],
    )
    def k(tbl_hbm, idx_hbm, out_hbm, tbl_v, idx_v, rows0, rows1, sem0, sem1):
        wid = lax.axis_index("s") * nc + lax.axis_index("c")
        i0w = pl.multiple_of(wid * i_per_w, i_per_w)
        pltpu.sync_copy(tbl_hbm, tbl_v)
        pltpu.sync_copy(idx_hbm.at[:, pl.ds(i0w, i_per_w)], idx_v)
        rows = [rows0, rows1]
        sems = [sem0, sem1]
        laneoff = lax.iota(jnp.int32, L)

        def make_copy(b, j):
            return pltpu.make_async_copy(
                rows[b],
                out_hbm.at[j, :, pl.ds(pl.multiple_of(i0w + b * _IC, _IC),
                                       _IC)],
                sems[b])

        def jloop(j, carry):
            for b in range(2):
                @pl.when(j > 0)
                def _wait(b=b, j=j):
                    make_copy(b, j - 1).wait()

                tmps = [
                    idx_v[j, pl.ds(b * _IC + g * L, L)] * L + laneoff
                    for g in range(groups)
                ]

                @plsc.parallel_loop(0, _D, unroll=1)
                def dloop(d, b=b):
                    doff = d * drow
                    for g in range(groups):
                        v = plsc.load_gather(tbl_v, [tmps[g] + doff])
                        rows[b][d, pl.ds(g * L, L)] = v

                make_copy(b, j).start()
            return carry

        lax.fori_loop(0, seq, jloop, 0)
        for b in range(2):
            make_copy(b, seq - 1).wait()

    return k(tbl_sk, idx_t)


def kernel(x, r, c, emb_input, emb_row, emb_col, W1, b1, W2, b2, W4, b4):
    del r, c, emb_row, emb_col
    n, s = x.shape
    emb_p = jnp.zeros((16, 16), jnp.float32).at[:emb_input.shape[0]].set(
        emb_input)
    table = _mlp_table(emb_p, W1, b1, W2, b2, W4, b4)
    tt = table.T
    tbl_sk = jnp.broadcast_to(tt[:, :, None], (_D, 16, 16))
    idx_t = x.astype(jnp.int32).T
    out_t = _sc_gather(tbl_sk.reshape(-1), idx_t, n, s)
    return jnp.transpose(out_t, (2, 0, 1))
